# baseline (speedup 1.0000x reference)
import functools
import jax
import jax.numpy as jnp
from jax import lax
from jax.experimental import pallas as pl
from jax.experimental.pallas import tpu as pltpu
from jax.experimental.pallas import tpu_sc as plsc

_N = 10000
_E = 320000
_H = 64
_HL = 32
_G = 64
_DE = 16
_BNS = 1.0 / (1.0 + 1e-5) ** 0.5

_RN = 2000
_EB = 4000

_NC = 2
_NS = 16
_NW = _NC * _NS
_GB = 128
_EBLK = _E // _GB
_NP = 2
_PB = _EBLK // _NP
_PE = _E // _NP
_WB = _PB // _NW
_XTRA = _PB - _WB * _NW
_K = 13
_NGRP = _WB // _K
_KS = 3
_NGS = _WB // _KS
_NRS = _N // _NS


def _inproj_body(x_ref, w_ref, b_ref, wd_ref, bd_ref, ws_ref,
                 h_ref, a_ref, b2_ref):
    h = jnp.dot(x_ref[...], w_ref[...], preferred_element_type=jnp.float32)
    h = h + b_ref[...]
    h_ref[...] = h
    a_ref[...] = (jnp.dot(h, wd_ref[...], preferred_element_type=jnp.float32)
                  + bd_ref[...]).astype(jnp.bfloat16)
    b2_ref[...] = jnp.dot(h, ws_ref[...],
                          preferred_element_type=jnp.float32).astype(jnp.bfloat16)


def _edge_body(ga_ref, gb_ref, ea_ref, we_ref, w2_ref, b2_ref, s2_ref,
               be2_ref, out_ref):
    t = (ga_ref[...].astype(jnp.float32) + gb_ref[...].astype(jnp.float32)
         + jnp.dot(ea_ref[...], we_ref[...],
                   preferred_element_type=jnp.float32))
    r = jnp.maximum(t, 0.0)
    m2 = jnp.dot(r, w2_ref[...], preferred_element_type=jnp.float32) + b2_ref[...]
    out_ref[...] = jnp.maximum(m2, 0.0) * s2_ref[...] + be2_ref[...]


def _update_body(h_ref, p00_ref, p01_ref, p10_ref, p11_ref,
                 w1h_ref, w1a_ref, b1_ref, w2_ref,
                 b2_ref, s2_ref, be2_ref, wd_ref, bd_ref, ws_ref,
                 h_out, a_out, b_out):
    h = h_ref[...]
    aggr = ((p00_ref[...] + p01_ref[...])
            + (p10_ref[...] + p11_ref[...]))
    t = (jnp.dot(h, w1h_ref[...], preferred_element_type=jnp.float32)
         + jnp.dot(aggr, w1a_ref[...], preferred_element_type=jnp.float32)
         + b1_ref[...])
    r = jnp.maximum(t, 0.0)
    u = jnp.dot(r, w2_ref[...], preferred_element_type=jnp.float32) + b2_ref[...]
    u = jnp.maximum(u, 0.0) * s2_ref[...] + be2_ref[...]
    hn = h + u
    h_out[...] = hn
    a_out[...] = (jnp.dot(hn, wd_ref[...], preferred_element_type=jnp.float32)
                  + bd_ref[...]).astype(jnp.bfloat16)
    b_out[...] = jnp.dot(hn, ws_ref[...],
                         preferred_element_type=jnp.float32).astype(jnp.bfloat16)


def _final_body(h_ref, p00_ref, p01_ref, p10_ref, p11_ref,
                w1h_ref, w1a_ref, b1_ref, w2_ref,
                b2_ref, s2_ref, be2_ref, batch_ref, ow_ref, ob_ref, out_ref):
    h = h_ref[...]
    aggr = ((p00_ref[...] + p01_ref[...])
            + (p10_ref[...] + p11_ref[...]))
    t = (jnp.dot(h, w1h_ref[...], preferred_element_type=jnp.float32)
         + jnp.dot(aggr, w1a_ref[...], preferred_element_type=jnp.float32)
         + b1_ref[...])
    r = jnp.maximum(t, 0.0)
    u = jnp.dot(r, w2_ref[...], preferred_element_type=jnp.float32) + b2_ref[...]
    u = jnp.maximum(u, 0.0) * s2_ref[...] + be2_ref[...]
    hn = h + u
    oh = (batch_ref[...] == lax.broadcasted_iota(jnp.int32, (1, _G), 1))
    ohf = oh.astype(jnp.float32)
    sums = lax.dot_general(ohf, hn, (((0,), (0,)), ((), ())),
                           preferred_element_type=jnp.float32)
    cnts = jnp.sum(ohf, axis=0)[:, None]
    mean = sums / jnp.maximum(cnts, 1.0)
    out_ref[...] = jnp.dot(mean, ow_ref[...],
                           preferred_element_type=jnp.float32) + ob_ref[...]


def _row_spec(bs, ncols):
    return pl.BlockSpec((bs, ncols), lambda i: (i, 0))


def _full_spec(shape):
    nd = len(shape)
    return pl.BlockSpec(shape, lambda i: (0,) * nd)


def _make_gather_body(pbase):
    def body(a_hbm, b_hbm, src2_hbm, dst2_hbm, ga_hbm, gb_hbm,
             idxd, idxs, bufa, bufb, xidx, sem_g, sem_w):
        c = lax.axis_index("c")
        s = lax.axis_index("s")
        w = s * _NC + c
        lrow0 = w * _WB
        pltpu.sync_copy(dst2_hbm.at[pl.ds(pbase + lrow0, _WB)], idxd)
        pltpu.sync_copy(src2_hbm.at[pl.ds(pbase + lrow0, _WB)], idxs)

        def grp(g, carry):
            k0 = g * _K
            cps = []
            for b in range(_K):
                cps.append(pltpu.async_copy(a_hbm.at[idxd.at[k0 + b]],
                                            bufa.at[b], sem_g))
                cps.append(pltpu.async_copy(b_hbm.at[idxs.at[k0 + b]],
                                            bufb.at[b], sem_g))
            for cp in cps:
                cp.wait()
            wbs = []
            for b in range(_K):
                off = (lrow0 + k0 + b) * _GB
                wbs.append(pltpu.async_copy(bufa.at[b],
                                            ga_hbm.at[pl.ds(off, _GB)], sem_w))
                wbs.append(pltpu.async_copy(bufb.at[b],
                                            gb_hbm.at[pl.ds(off, _GB)], sem_w))
            for cp in wbs:
                cp.wait()
            return carry

        lax.fori_loop(0, _NGRP, grp, 0)

        @pl.when(w < _XTRA)
        def _():
            lj = _NW * _WB + w
            pltpu.sync_copy(dst2_hbm.at[pl.ds(pbase + lj, 1)],
                            xidx.at[pl.ds(0, 1)])
            pltpu.sync_copy(src2_hbm.at[pl.ds(pbase + lj, 1)],
                            xidx.at[pl.ds(1, 1)])
            cp_a = pltpu.async_copy(a_hbm.at[xidx.at[0]], bufa.at[0], sem_g)
            cp_b = pltpu.async_copy(b_hbm.at[xidx.at[1]], bufb.at[0], sem_g)
            cp_a.wait()
            cp_b.wait()
            off = lj * _GB
            pltpu.sync_copy(bufa.at[0], ga_hbm.at[pl.ds(off, _GB)])
            pltpu.sync_copy(bufb.at[0], gb_hbm.at[pl.ds(off, _GB)])

    return body


def _make_scatter_body(pbase):
    def body(eo_hbm, dst2_hbm, z_hbm, out_hbm,
             idxd, xidx, rows_v, acc_sh, sem_l):
        c = lax.axis_index("c")
        s = lax.axis_index("s")
        w = s * _NC + c
        lrow0 = w * _WB
        r0 = s * _NRS
        pltpu.sync_copy(z_hbm.at[pl.ds(r0, _NRS)], acc_sh.at[pl.ds(r0, _NRS)])
        pltpu.sync_copy(dst2_hbm.at[pl.ds(pbase + lrow0, _WB)], idxd)
        plsc.subcore_barrier()

        def grp(g, carry):
            k0 = g * _KS
            cps = []
            for b in range(_KS):
                off = (lrow0 + k0 + b) * _GB
                cps.append(pltpu.async_copy(eo_hbm.at[pl.ds(off, _GB)],
                                            rows_v.at[b], sem_l))
            for b in range(_KS):
                cps[b].wait()
                pltpu.sync_copy(rows_v.at[b], acc_sh.at[idxd.at[k0 + b]],
                                add=True)
            return carry

        lax.fori_loop(0, _NGS, grp, 0)

        @pl.when(w < _XTRA)
        def _():
            lj = _NW * _WB + w
            pltpu.sync_copy(dst2_hbm.at[pl.ds(pbase + lj, 1)],
                            xidx.at[pl.ds(0, 1)])
            pltpu.sync_copy(eo_hbm.at[pl.ds(lj * _GB, _GB)], rows_v.at[0])
            pltpu.sync_copy(rows_v.at[0], acc_sh.at[xidx.at[0]], add=True)

        plsc.subcore_barrier()
        pltpu.sync_copy(acc_sh.at[pl.ds(r0, _NRS)],
                        out_hbm.at[pl.ds(c * _N + r0, _NRS)])

    return body


def kernel(x, edge_index, edge_attr, batch, lin_in_W, lin_in_b, msg_W1, msg_b1,
           msg_g1, msg_be1, msg_W2, msg_b2, msg_g2, msg_be2, upd_W1, upd_b1,
           upd_g1, upd_be1, upd_W2, upd_b2, upd_g2, upd_be2, out_W, out_b):
    L = msg_W1.shape[0]
    src = edge_index[0]
    dst = edge_index[1]

    ms1 = msg_g1 * _BNS
    ms2 = msg_g2 * _BNS
    us1 = upd_g1 * _BNS
    us2 = upd_g2 * _BNS
    mWd = msg_W1[:, :_H, :] * ms1[:, None, :]
    mWs = msg_W1[:, _H:2 * _H, :] * ms1[:, None, :]
    mWe = msg_W1[:, 2 * _H:, :] * ms1[:, None, :]
    mbd = msg_b1 * ms1 + msg_be1
    uWh = upd_W1[:, :_H, :] * us1[:, None, :]
    uWa = upd_W1[:, _H:, :] * us1[:, None, :]
    ub1 = upd_b1 * us1 + upd_be1

    f32 = jnp.float32
    zeros_n = jnp.zeros((_N, _H), dtype=f32)

    inproj = pl.pallas_call(
        _inproj_body,
        grid=(_N // _RN,),
        in_specs=[_row_spec(_RN, 128), _full_spec((128, _H)), _full_spec((1, _H)),
                  _full_spec((_H, _HL)), _full_spec((1, _HL)), _full_spec((_H, _HL))],
        out_specs=[_row_spec(_RN, _H), _row_spec(_RN, _HL), _row_spec(_RN, _HL)],
        out_shape=[jax.ShapeDtypeStruct((_N, _H), f32),
                   jax.ShapeDtypeStruct((_N, _HL), jnp.bfloat16),
                   jax.ShapeDtypeStruct((_N, _HL), jnp.bfloat16)],
    )
    h, A, B = inproj(x, lin_in_W, lin_in_b.reshape(1, _H),
                     mWd[0], mbd[0].reshape(1, _HL), mWs[0])

    edge_mlp = pl.pallas_call(
        _edge_body,
        grid=(_PE // _EB,),
        in_specs=[_row_spec(_EB, _HL), _row_spec(_EB, _HL), _row_spec(_EB, _DE),
                  _full_spec((_DE, _HL)), _full_spec((_HL, _H)),
                  _full_spec((1, _H)), _full_spec((1, _H)), _full_spec((1, _H))],
        out_specs=_row_spec(_EB, _H),
        out_shape=jax.ShapeDtypeStruct((_PE, _H), f32),
    )

    update = pl.pallas_call(
        _update_body,
        grid=(_N // _RN,),
        in_specs=[_row_spec(_RN, _H),
                  _row_spec(_RN, _H), _row_spec(_RN, _H),
                  _row_spec(_RN, _H), _row_spec(_RN, _H),
                  _full_spec((_H, _HL)), _full_spec((_H, _HL)), _full_spec((1, _HL)),
                  _full_spec((_HL, _H)), _full_spec((1, _H)), _full_spec((1, _H)),
                  _full_spec((1, _H)),
                  _full_spec((_H, _HL)), _full_spec((1, _HL)), _full_spec((_H, _HL))],
        out_specs=[_row_spec(_RN, _H), _row_spec(_RN, _HL), _row_spec(_RN, _HL)],
        out_shape=[jax.ShapeDtypeStruct((_N, _H), f32),
                   jax.ShapeDtypeStruct((_N, _HL), jnp.bfloat16),
                   jax.ShapeDtypeStruct((_N, _HL), jnp.bfloat16)],
    )

    final = pl.pallas_call(
        _final_body,
        grid=(1,),
        in_specs=[_full_spec((_N, _H)),
                  _full_spec((_N, _H)), _full_spec((_N, _H)),
                  _full_spec((_N, _H)), _full_spec((_N, _H)),
                  _full_spec((_H, _HL)), _full_spec((_H, _HL)), _full_spec((1, _HL)),
                  _full_spec((_HL, _H)), _full_spec((1, _H)), _full_spec((1, _H)),
                  _full_spec((1, _H)),
                  _full_spec((_N, 1)), _full_spec((_H, 1)), _full_spec((1, 1))],
        out_specs=_full_spec((_G, 1)),
        out_shape=jax.ShapeDtypeStruct((_G, 1), f32),
    )

    mesh = plsc.VectorSubcoreMesh(core_axis_name="c", subcore_axis_name="s")
    scp = pltpu.CompilerParams(use_tc_tiling_on_sc=False)

    gather_scratch = [pltpu.VMEM((_WB, _GB), jnp.int32),
                      pltpu.VMEM((_WB, _GB), jnp.int32),
                      pltpu.VMEM((_K, _GB, _HL), jnp.bfloat16),
                      pltpu.VMEM((_K, _GB, _HL), jnp.bfloat16),
                      pltpu.VMEM((2, _GB), jnp.int32),
                      pltpu.SemaphoreType.DMA,
                      pltpu.SemaphoreType.DMA]
    scatter_scratch = [pltpu.VMEM((_WB, _GB), jnp.int32),
                       pltpu.VMEM((2, _GB), jnp.int32),
                       pltpu.VMEM((_KS, _GB, _H), f32),
                       pltpu.VMEM_SHARED((_N, _H), f32),
                       pltpu.SemaphoreType.DMA]

    gather_p = []
    scatter_p = []
    for p in range(_NP):
        gather_p.append(pl.kernel(
            _make_gather_body(p * _PB),
            out_type=[jax.ShapeDtypeStruct((_PE, _HL), jnp.bfloat16),
                      jax.ShapeDtypeStruct((_PE, _HL), jnp.bfloat16)],
            scratch_types=gather_scratch,
            mesh=mesh,
            compiler_params=scp,
        ))
        scatter_p.append(pl.kernel(
            _make_scatter_body(p * _PB),
            out_type=jax.ShapeDtypeStruct((_NC * _N, _H), f32),
            scratch_types=scatter_scratch,
            mesh=mesh,
            compiler_params=scp,
        ))

    src2 = src.reshape(_EBLK, _GB)
    dst2 = dst.reshape(_EBLK, _GB)
    ea_p = [edge_attr[p * _PE:(p + 1) * _PE] for p in range(_NP)]

    for l in range(L):
        parts = []
        for p in range(_NP):
            gA, gB = gather_p[p](A, B, src2, dst2)
            eout = edge_mlp(gA, gB, ea_p[p], mWe[l], msg_W2[l],
                            msg_b2[l].reshape(1, _H), ms2[l].reshape(1, _H),
                            msg_be2[l].reshape(1, _H))
            parts.append(scatter_p[p](eout, dst2, zeros_n))
        p00 = parts[0][:_N]
        p01 = parts[0][_N:]
        p10 = parts[1][:_N]
        p11 = parts[1][_N:]
        if l + 1 < L:
            h, A, B = update(h, p00, p01, p10, p11,
                             uWh[l], uWa[l], ub1[l].reshape(1, _HL),
                             upd_W2[l], upd_b2[l].reshape(1, _H),
                             us2[l].reshape(1, _H), upd_be2[l].reshape(1, _H),
                             mWd[l + 1], mbd[l + 1].reshape(1, _HL), mWs[l + 1])
        else:
            out = final(h, p00, p01, p10, p11,
                        uWh[l], uWa[l], ub1[l].reshape(1, _HL),
                        upd_W2[l], upd_b2[l].reshape(1, _H),
                        us2[l].reshape(1, _H), upd_be2[l].reshape(1, _H),
                        batch.reshape(_N, 1), out_W, out_b.reshape(1, 1))
    return out.reshape(-1)


# speedup vs baseline: 1.9173x; 1.9173x over previous
import functools
import jax
import jax.numpy as jnp
from jax import lax
from jax.experimental import pallas as pl
from jax.experimental.pallas import tpu as pltpu
from jax.experimental.pallas import tpu_sc as plsc

_N = 10000
_E = 320000
_H = 64
_HL = 32
_G = 64
_DE = 16
_BNS = 1.0 / (1.0 + 1e-5) ** 0.5

_RN = 2000
_EB = 3200
_BB = _EB // 8

_NC = 2
_NS = 16
_NW = _NC * _NS
_GB = 128
_GR = _GB // 8
_EBLK = _E // _GB
_NP = 2
_PB = _EBLK // _NP
_PE = _E // _NP
_WB = _PB // _NW
_XTRA = _PB - _WB * _NW
_K = 6
_NGRP = _WB // _K
_EPI = _WB - _NGRP * _K
_KS = 3
_NGS = _WB // _KS
_NRS = _N // _NS


def _inproj_body(x_ref, w_ref, b_ref, wd_ref, bd_ref, ws_ref,
                 h_ref, a_ref, b2_ref):
    h = jnp.dot(x_ref[...], w_ref[...], preferred_element_type=jnp.float32)
    h = h + b_ref[...]
    h_ref[...] = h
    a_ref[...] = jnp.dot(h, wd_ref[...],
                         preferred_element_type=jnp.float32) + bd_ref[...]
    b2_ref[...] = jnp.dot(h, ws_ref[...], preferred_element_type=jnp.float32)


def _edge_body(g_ref, ea_ref, wep_ref, w2p_ref, b2_ref, s2_ref,
               be2_ref, out_ref):
    t = g_ref[...] + jnp.dot(ea_ref[...], wep_ref[...],
                             preferred_element_type=jnp.float32)
    r = jnp.maximum(t, 0.0)
    m2 = jnp.dot(r, w2p_ref[...], preferred_element_type=jnp.float32) + b2_ref[...]
    out_ref[...] = jnp.maximum(m2, 0.0) * s2_ref[...] + be2_ref[...]


def _update_body(h_ref, p00_ref, p01_ref, p10_ref, p11_ref,
                 w1h_ref, w1a_ref, b1_ref, w2_ref,
                 b2_ref, s2_ref, be2_ref, wd_ref, bd_ref, ws_ref,
                 h_out, a_out, b_out):
    h = h_ref[...]
    aggr = ((p00_ref[...] + p01_ref[...])
            + (p10_ref[...] + p11_ref[...]))
    t = (jnp.dot(h, w1h_ref[...], preferred_element_type=jnp.float32)
         + jnp.dot(aggr, w1a_ref[...], preferred_element_type=jnp.float32)
         + b1_ref[...])
    r = jnp.maximum(t, 0.0)
    u = jnp.dot(r, w2_ref[...], preferred_element_type=jnp.float32) + b2_ref[...]
    u = jnp.maximum(u, 0.0) * s2_ref[...] + be2_ref[...]
    hn = h + u
    h_out[...] = hn
    a_out[...] = jnp.dot(hn, wd_ref[...],
                         preferred_element_type=jnp.float32) + bd_ref[...]
    b_out[...] = jnp.dot(hn, ws_ref[...], preferred_element_type=jnp.float32)


def _final_body(h_ref, p00_ref, p01_ref, p10_ref, p11_ref,
                w1h_ref, w1a_ref, b1_ref, w2_ref,
                b2_ref, s2_ref, be2_ref, batch_ref, ow_ref, ob_ref, out_ref):
    h = h_ref[...]
    aggr = ((p00_ref[...] + p01_ref[...])
            + (p10_ref[...] + p11_ref[...]))
    t = (jnp.dot(h, w1h_ref[...], preferred_element_type=jnp.float32)
         + jnp.dot(aggr, w1a_ref[...], preferred_element_type=jnp.float32)
         + b1_ref[...])
    r = jnp.maximum(t, 0.0)
    u = jnp.dot(r, w2_ref[...], preferred_element_type=jnp.float32) + b2_ref[...]
    u = jnp.maximum(u, 0.0) * s2_ref[...] + be2_ref[...]
    hn = h + u
    oh = (batch_ref[...] == lax.broadcasted_iota(jnp.int32, (1, _G), 1))
    ohf = oh.astype(jnp.float32)
    sums = lax.dot_general(ohf, hn, (((0,), (0,)), ((), ())),
                           preferred_element_type=jnp.float32)
    cnts = jnp.sum(ohf, axis=0)[:, None]
    mean = sums / jnp.maximum(cnts, 1.0)
    out_ref[...] = jnp.dot(mean, ow_ref[...],
                           preferred_element_type=jnp.float32) + ob_ref[...]


def _row_spec(bs, ncols):
    return pl.BlockSpec((bs, ncols), lambda i: (i, 0))


def _full_spec(shape):
    nd = len(shape)
    return pl.BlockSpec(shape, lambda i: (0,) * nd)


def _make_gather_body(pbase):
    def body(a_hbm, b_hbm, src2_hbm, dst2_hbm, g8_hbm,
             idxd, idxs, bufa, bufb, bufg, xidx, sem_g, sem_w):
        c = lax.axis_index("c")
        s = lax.axis_index("s")
        w = s * _NC + c
        lrow0 = w * _WB
        pltpu.sync_copy(dst2_hbm.at[pl.ds(pbase + lrow0, _WB)], idxd)
        pltpu.sync_copy(src2_hbm.at[pl.ds(pbase + lrow0, _WB)], idxs)

        def addpack(b):
            def rowfn(r, carry):
                for q in range(8):
                    e = r * 8 + q
                    for u in range(2):
                        bufg[b, r, pl.ds(q * 32 + u * 16, 16)] = (
                            bufa[b, e, pl.ds(u * 16, 16)]
                            + bufb[b, e, pl.ds(u * 16, 16)])
                return carry
            lax.fori_loop(0, _GR, rowfn, 0)

        def emit_group(k0, cnt):
            cps = []
            for b in range(cnt):
                cps.append(pltpu.async_copy(a_hbm.at[idxd.at[k0 + b]],
                                            bufa.at[b], sem_g))
                cps.append(pltpu.async_copy(b_hbm.at[idxs.at[k0 + b]],
                                            bufb.at[b], sem_g))
            wbs = []
            for b in range(cnt):
                cps[2 * b].wait()
                cps[2 * b + 1].wait()
                addpack(b)
                off = (lrow0 + k0 + b) * _GR
                wbs.append(pltpu.async_copy(bufg.at[b],
                                            g8_hbm.at[pl.ds(off, _GR)],
                                            sem_w))
            for cp in wbs:
                cp.wait()

        def grp(g, carry):
            emit_group(g * _K, _K)
            return carry

        lax.fori_loop(0, _NGRP, grp, 0)
        emit_group(_NGRP * _K, _EPI)

        @pl.when(w < _XTRA)
        def _():
            lj = _NW * _WB + w
            pltpu.sync_copy(dst2_hbm.at[pl.ds(pbase + lj, 1)],
                            xidx.at[pl.ds(0, 1)])
            pltpu.sync_copy(src2_hbm.at[pl.ds(pbase + lj, 1)],
                            xidx.at[pl.ds(1, 1)])
            cp_a = pltpu.async_copy(a_hbm.at[xidx.at[0]], bufa.at[0], sem_g)
            cp_b = pltpu.async_copy(b_hbm.at[xidx.at[1]], bufb.at[0], sem_g)
            cp_a.wait()
            cp_b.wait()
            addpack(0)
            pltpu.sync_copy(bufg.at[0], g8_hbm.at[pl.ds(lj * _GR, _GR)])

    return body


def _make_scatter_body(pbase):
    def body(eo_hbm, dst2_hbm, z_hbm, out_hbm,
             idxd, xidx, rows_v, acc_sh, sem_l):
        c = lax.axis_index("c")
        s = lax.axis_index("s")
        w = s * _NC + c
        lrow0 = w * _WB
        r0 = s * _NRS
        pltpu.sync_copy(z_hbm.at[pl.ds(r0, _NRS)], acc_sh.at[pl.ds(r0, _NRS)])
        pltpu.sync_copy(dst2_hbm.at[pl.ds(pbase + lrow0, _WB)], idxd)
        plsc.subcore_barrier()

        def grp(g, carry):
            k0 = g * _KS
            cps = []
            for b in range(_KS):
                off = (lrow0 + k0 + b) * _GB
                cps.append(pltpu.async_copy(eo_hbm.at[pl.ds(off, _GB)],
                                            rows_v.at[b], sem_l))
            for b in range(_KS):
                cps[b].wait()
                pltpu.sync_copy(rows_v.at[b], acc_sh.at[idxd.at[k0 + b]],
                                add=True)
            return carry

        lax.fori_loop(0, _NGS, grp, 0)

        @pl.when(w < _XTRA)
        def _():
            lj = _NW * _WB + w
            pltpu.sync_copy(dst2_hbm.at[pl.ds(pbase + lj, 1)],
                            xidx.at[pl.ds(0, 1)])
            pltpu.sync_copy(eo_hbm.at[pl.ds(lj * _GB, _GB)], rows_v.at[0])
            pltpu.sync_copy(rows_v.at[0], acc_sh.at[xidx.at[0]], add=True)

        plsc.subcore_barrier()
        pltpu.sync_copy(acc_sh.at[pl.ds(r0, _NRS)],
                        out_hbm.at[pl.ds(c * _N + r0, _NRS)])

    return body


def _blockdiag(w, k):
    a, b = w.shape
    out = jnp.zeros((k * a, k * b), dtype=w.dtype)
    for i in range(k):
        out = out.at[i * a:(i + 1) * a, i * b:(i + 1) * b].set(w)
    return out


def kernel(x, edge_index, edge_attr, batch, lin_in_W, lin_in_b, msg_W1, msg_b1,
           msg_g1, msg_be1, msg_W2, msg_b2, msg_g2, msg_be2, upd_W1, upd_b1,
           upd_g1, upd_be1, upd_W2, upd_b2, upd_g2, upd_be2, out_W, out_b):
    L = msg_W1.shape[0]
    src = edge_index[0]
    dst = edge_index[1]

    ms1 = msg_g1 * _BNS
    ms2 = msg_g2 * _BNS
    us1 = upd_g1 * _BNS
    us2 = upd_g2 * _BNS
    mWd = msg_W1[:, :_H, :] * ms1[:, None, :]
    mWs = msg_W1[:, _H:2 * _H, :] * ms1[:, None, :]
    mWe = msg_W1[:, 2 * _H:, :] * ms1[:, None, :]
    mbd = msg_b1 * ms1 + msg_be1
    uWh = upd_W1[:, :_H, :] * us1[:, None, :]
    uWa = upd_W1[:, _H:, :] * us1[:, None, :]
    ub1 = upd_b1 * us1 + upd_be1

    f32 = jnp.float32
    zeros_n = jnp.zeros((_N, _H), dtype=f32)

    wep = [_blockdiag(mWe[l], 8) for l in range(L)]
    w2p = [_blockdiag(msg_W2[l], 8) for l in range(L)]
    b2t = [jnp.tile(msg_b2[l], 8).reshape(1, 8 * _H) for l in range(L)]
    s2t = [jnp.tile(ms2[l], 8).reshape(1, 8 * _H) for l in range(L)]
    be2t = [jnp.tile(msg_be2[l], 8).reshape(1, 8 * _H) for l in range(L)]

    inproj = pl.pallas_call(
        _inproj_body,
        grid=(_N // _RN,),
        in_specs=[_row_spec(_RN, 128), _full_spec((128, _H)), _full_spec((1, _H)),
                  _full_spec((_H, _HL)), _full_spec((1, _HL)), _full_spec((_H, _HL))],
        out_specs=[_row_spec(_RN, _H), _row_spec(_RN, _HL), _row_spec(_RN, _HL)],
        out_shape=[jax.ShapeDtypeStruct((_N, _H), f32),
                   jax.ShapeDtypeStruct((_N, _HL), f32),
                   jax.ShapeDtypeStruct((_N, _HL), f32)],
    )
    h, A, B = inproj(x, lin_in_W, lin_in_b.reshape(1, _H),
                     mWd[0], mbd[0].reshape(1, _HL), mWs[0])

    edge_mlp_p = []
    for p in range(_NP):
        poff = p * (_PE // _EB)
        edge_mlp_p.append(pl.pallas_call(
            _edge_body,
            grid=(_PE // _EB,),
            in_specs=[_row_spec(_BB, 8 * _HL),
                      pl.BlockSpec((_BB, 128), lambda i, poff=poff: (i + poff, 0)),
                      _full_spec((128, 8 * _HL)), _full_spec((8 * _HL, 8 * _H)),
                      _full_spec((1, 8 * _H)), _full_spec((1, 8 * _H)),
                      _full_spec((1, 8 * _H))],
            out_specs=_row_spec(_BB, 8 * _H),
            out_shape=jax.ShapeDtypeStruct((_PE // 8, 8 * _H), f32),
        ))

    def _pspec(j):
        return pl.BlockSpec((_RN, _H), lambda i, j=j: (i + j * (_N // _RN), 0))

    update = pl.pallas_call(
        _update_body,
        grid=(_N // _RN,),
        in_specs=[_row_spec(_RN, _H),
                  _pspec(0), _pspec(1), _pspec(0), _pspec(1),
                  _full_spec((_H, _HL)), _full_spec((_H, _HL)), _full_spec((1, _HL)),
                  _full_spec((_HL, _H)), _full_spec((1, _H)), _full_spec((1, _H)),
                  _full_spec((1, _H)),
                  _full_spec((_H, _HL)), _full_spec((1, _HL)), _full_spec((_H, _HL))],
        out_specs=[_row_spec(_RN, _H), _row_spec(_RN, _HL), _row_spec(_RN, _HL)],
        out_shape=[jax.ShapeDtypeStruct((_N, _H), f32),
                   jax.ShapeDtypeStruct((_N, _HL), f32),
                   jax.ShapeDtypeStruct((_N, _HL), f32)],
    )

    def _fpspec(j):
        return pl.BlockSpec((_N, _H), lambda i, j=j: (j, 0))

    final = pl.pallas_call(
        _final_body,
        grid=(1,),
        in_specs=[_full_spec((_N, _H)),
                  _fpspec(0), _fpspec(1), _fpspec(0), _fpspec(1),
                  _full_spec((_H, _HL)), _full_spec((_H, _HL)), _full_spec((1, _HL)),
                  _full_spec((_HL, _H)), _full_spec((1, _H)), _full_spec((1, _H)),
                  _full_spec((1, _H)),
                  _full_spec((_N, 1)), _full_spec((_H, 1)), _full_spec((1, 1))],
        out_specs=_full_spec((_G, 1)),
        out_shape=jax.ShapeDtypeStruct((_G, 1), f32),
    )

    mesh = plsc.VectorSubcoreMesh(core_axis_name="c", subcore_axis_name="s")
    scp = pltpu.CompilerParams(use_tc_tiling_on_sc=False)

    gather_scratch = [pltpu.VMEM((_WB, _GB), jnp.int32),
                      pltpu.VMEM((_WB, _GB), jnp.int32),
                      pltpu.VMEM((_K, _GB, _HL), f32),
                      pltpu.VMEM((_K, _GB, _HL), f32),
                      pltpu.VMEM((_K, _GR, 8 * _HL), f32),
                      pltpu.VMEM((2, _GB), jnp.int32),
                      pltpu.SemaphoreType.DMA,
                      pltpu.SemaphoreType.DMA]
    scatter_scratch = [pltpu.VMEM((_WB, _GB), jnp.int32),
                       pltpu.VMEM((2, _GB), jnp.int32),
                       pltpu.VMEM((_KS, _GB, _H), f32),
                       pltpu.VMEM_SHARED((_N, _H), f32),
                       pltpu.SemaphoreType.DMA]

    gather_p = []
    scatter_p = []
    for p in range(_NP):
        gather_p.append(pl.kernel(
            _make_gather_body(p * _PB),
            out_type=jax.ShapeDtypeStruct((_PE // 8, 8 * _HL), f32),
            scratch_types=gather_scratch,
            mesh=mesh,
            compiler_params=scp,
        ))
        scatter_p.append(pl.kernel(
            _make_scatter_body(p * _PB),
            out_type=jax.ShapeDtypeStruct((_NC * _N, _H), f32),
            scratch_types=scatter_scratch,
            mesh=mesh,
            compiler_params=scp,
        ))

    src2 = src.reshape(_EBLK, _GB)
    dst2 = dst.reshape(_EBLK, _GB)
    ea8 = edge_attr.reshape(_E // 8, 8 * _DE)

    for l in range(L):
        parts = []
        for p in range(_NP):
            g8 = gather_p[p](A, B, src2, dst2)
            eout8 = edge_mlp_p[p](g8, ea8, wep[l], w2p[l],
                                  b2t[l], s2t[l], be2t[l])
            eout = eout8.reshape(_PE, _H)
            parts.append(scatter_p[p](eout, dst2, zeros_n))
        if l + 1 < L:
            h, A, B = update(h, parts[0], parts[0], parts[1], parts[1],
                             uWh[l], uWa[l], ub1[l].reshape(1, _HL),
                             upd_W2[l], upd_b2[l].reshape(1, _H),
                             us2[l].reshape(1, _H), upd_be2[l].reshape(1, _H),
                             mWd[l + 1], mbd[l + 1].reshape(1, _HL), mWs[l + 1])
        else:
            out = final(h, parts[0], parts[0], parts[1], parts[1],
                        uWh[l], uWa[l], ub1[l].reshape(1, _HL),
                        upd_W2[l], upd_b2[l].reshape(1, _H),
                        us2[l].reshape(1, _H), upd_be2[l].reshape(1, _H),
                        batch.reshape(_N, 1), out_W, out_b.reshape(1, 1))
    return out.reshape(-1)


# speedup vs baseline: 2.4699x; 1.2882x over previous
import functools
import jax
import jax.numpy as jnp
from jax import lax
from jax.experimental import pallas as pl
from jax.experimental.pallas import tpu as pltpu
from jax.experimental.pallas import tpu_sc as plsc

_N = 10000
_E = 320000
_H = 64
_HL = 32
_G = 64
_DE = 16
_BNS = 1.0 / (1.0 + 1e-5) ** 0.5

_RN = 2000
_EB = 3200
_BB = _EB // 8

_NC = 2
_NS = 16
_NW = _NC * _NS
_GB = 128
_GR = _GB // 8
_EBLK = _E // _GB
_NP = 2
_PB = _EBLK // _NP
_PE = _E // _NP
_WB = _PB // _NW
_XTRA = _PB - _WB * _NW
_K = 6
_NGRP = _WB // _K
_EPI = _WB - _NGRP * _K
_KS = 3
_NGS = _WB // _KS
_NRS = _N // _NS


def _inproj_body(x_ref, w_ref, b_ref, wd_ref, bd_ref, ws_ref,
                 h_ref, a_ref, b2_ref):
    h = jnp.dot(x_ref[...], w_ref[...], preferred_element_type=jnp.float32)
    h = h + b_ref[...]
    h_ref[...] = h
    a_ref[...] = jnp.dot(h, wd_ref[...],
                         preferred_element_type=jnp.float32) + bd_ref[...]
    b2_ref[...] = jnp.dot(h, ws_ref[...], preferred_element_type=jnp.float32)


def _edge_body(g_ref, ea_ref, wep_ref, w2p_ref, b2_ref, s2_ref,
               be2_ref, out_ref):
    t = g_ref[...] + jnp.dot(ea_ref[...], wep_ref[...],
                             preferred_element_type=jnp.float32)
    r = jnp.maximum(t, 0.0)
    m2 = jnp.dot(r, w2p_ref[...], preferred_element_type=jnp.float32) + b2_ref[...]
    o = jnp.maximum(m2, 0.0) * s2_ref[...] + be2_ref[...]
    out_ref[...] = o.reshape(_EB // 2, 2 * _H)


def _update_body(h_ref, p00_ref, p01_ref, p10_ref, p11_ref,
                 w1h_ref, w1a_ref, b1_ref, w2_ref,
                 b2_ref, s2_ref, be2_ref, wd_ref, bd_ref, ws_ref,
                 h_out, a_out, b_out):
    h = h_ref[...]
    aggr = ((p00_ref[...] + p01_ref[...])
            + (p10_ref[...] + p11_ref[...]))
    t = (jnp.dot(h, w1h_ref[...], preferred_element_type=jnp.float32)
         + jnp.dot(aggr, w1a_ref[...], preferred_element_type=jnp.float32)
         + b1_ref[...])
    r = jnp.maximum(t, 0.0)
    u = jnp.dot(r, w2_ref[...], preferred_element_type=jnp.float32) + b2_ref[...]
    u = jnp.maximum(u, 0.0) * s2_ref[...] + be2_ref[...]
    hn = h + u
    h_out[...] = hn
    a_out[...] = jnp.dot(hn, wd_ref[...],
                         preferred_element_type=jnp.float32) + bd_ref[...]
    b_out[...] = jnp.dot(hn, ws_ref[...], preferred_element_type=jnp.float32)


def _final_body(h_ref, p00_ref, p01_ref, p10_ref, p11_ref,
                w1h_ref, w1a_ref, b1_ref, w2_ref,
                b2_ref, s2_ref, be2_ref, batch_ref, ow_ref, ob_ref, out_ref):
    h = h_ref[...]
    aggr = ((p00_ref[...] + p01_ref[...])
            + (p10_ref[...] + p11_ref[...]))
    t = (jnp.dot(h, w1h_ref[...], preferred_element_type=jnp.float32)
         + jnp.dot(aggr, w1a_ref[...], preferred_element_type=jnp.float32)
         + b1_ref[...])
    r = jnp.maximum(t, 0.0)
    u = jnp.dot(r, w2_ref[...], preferred_element_type=jnp.float32) + b2_ref[...]
    u = jnp.maximum(u, 0.0) * s2_ref[...] + be2_ref[...]
    hn = h + u
    oh = (batch_ref[...] == lax.broadcasted_iota(jnp.int32, (1, _G), 1))
    ohf = oh.astype(jnp.float32)
    sums = lax.dot_general(ohf, hn, (((0,), (0,)), ((), ())),
                           preferred_element_type=jnp.float32)
    cnts = jnp.sum(ohf, axis=0)[:, None]
    mean = sums / jnp.maximum(cnts, 1.0)
    out_ref[...] = jnp.dot(mean, ow_ref[...],
                           preferred_element_type=jnp.float32) + ob_ref[...]


def _row_spec(bs, ncols):
    return pl.BlockSpec((bs, ncols), lambda i: (i, 0))


def _full_spec(shape):
    nd = len(shape)
    return pl.BlockSpec(shape, lambda i: (0,) * nd)


def _make_gather_body(pbase):
    def body(a_hbm, b_hbm, src2_hbm, dst2_hbm, g8_hbm,
             idxd, idxs, bufa, bufb, bufg, xidx, sem_g, sem_w):
        c = lax.axis_index("c")
        s = lax.axis_index("s")
        w = s * _NC + c
        lrow0 = w * _WB
        pltpu.sync_copy(dst2_hbm.at[pl.ds(pbase + lrow0, _WB)], idxd)
        pltpu.sync_copy(src2_hbm.at[pl.ds(pbase + lrow0, _WB)], idxs)

        def addpack(b):
            def rowfn(r, carry):
                for q in range(8):
                    e = r * 8 + q
                    for u in range(2):
                        bufg[b, r, pl.ds(q * 32 + u * 16, 16)] = (
                            bufa[b, e, pl.ds(u * 16, 16)]
                            + bufb[b, e, pl.ds(u * 16, 16)])
                return carry
            lax.fori_loop(0, _GR, rowfn, 0)

        def emit_group(k0, cnt):
            cps = []
            for b in range(cnt):
                cps.append(pltpu.async_copy(a_hbm.at[idxd.at[k0 + b]],
                                            bufa.at[b], sem_g))
                cps.append(pltpu.async_copy(b_hbm.at[idxs.at[k0 + b]],
                                            bufb.at[b], sem_g))
            wbs = []
            for b in range(cnt):
                cps[2 * b].wait()
                cps[2 * b + 1].wait()
                addpack(b)
                off = (lrow0 + k0 + b) * _GR
                wbs.append(pltpu.async_copy(bufg.at[b],
                                            g8_hbm.at[pl.ds(off, _GR)],
                                            sem_w))
            for cp in wbs:
                cp.wait()

        def grp(g, carry):
            emit_group(g * _K, _K)
            return carry

        lax.fori_loop(0, _NGRP, grp, 0)
        emit_group(_NGRP * _K, _EPI)

        @pl.when(w < _XTRA)
        def _():
            lj = _NW * _WB + w
            pltpu.sync_copy(dst2_hbm.at[pl.ds(pbase + lj, 1)],
                            xidx.at[pl.ds(0, 1)])
            pltpu.sync_copy(src2_hbm.at[pl.ds(pbase + lj, 1)],
                            xidx.at[pl.ds(1, 1)])
            cp_a = pltpu.async_copy(a_hbm.at[xidx.at[0]], bufa.at[0], sem_g)
            cp_b = pltpu.async_copy(b_hbm.at[xidx.at[1]], bufb.at[0], sem_g)
            cp_a.wait()
            cp_b.wait()
            addpack(0)
            pltpu.sync_copy(bufg.at[0], g8_hbm.at[pl.ds(lj * _GR, _GR)])

    return body


def _make_scatter_body(pbase):
    def body(eo_hbm, dst2_hbm, z_hbm, out_hbm,
             idxd, xidx, rows_v, acc_sh, sem_l):
        c = lax.axis_index("c")
        s = lax.axis_index("s")
        w = s * _NC + c
        lrow0 = w * _WB
        r0 = s * _NRS
        pltpu.sync_copy(z_hbm.at[pl.ds(r0, _NRS)], acc_sh.at[pl.ds(r0, _NRS)])
        pltpu.sync_copy(dst2_hbm.at[pl.ds(pbase + lrow0, _WB)], idxd)
        plsc.subcore_barrier()

        def grp(g, carry):
            k0 = g * _KS
            cps = []
            for b in range(_KS):
                off = (lrow0 + k0 + b) * _GB
                cps.append(pltpu.async_copy(eo_hbm.at[pl.ds(off, _GB)],
                                            rows_v.at[b], sem_l))
            for b in range(_KS):
                cps[b].wait()
                pltpu.sync_copy(rows_v.at[b], acc_sh.at[idxd.at[k0 + b]],
                                add=True)
            return carry

        lax.fori_loop(0, _NGS, grp, 0)

        @pl.when(w < _XTRA)
        def _():
            lj = _NW * _WB + w
            pltpu.sync_copy(dst2_hbm.at[pl.ds(pbase + lj, 1)],
                            xidx.at[pl.ds(0, 1)])
            pltpu.sync_copy(eo_hbm.at[pl.ds(lj * _GB, _GB)], rows_v.at[0])
            pltpu.sync_copy(rows_v.at[0], acc_sh.at[xidx.at[0]], add=True)

        plsc.subcore_barrier()
        pltpu.sync_copy(acc_sh.at[pl.ds(r0, _NRS)],
                        out_hbm.at[pl.ds(c * _N + r0, _NRS)])

    return body


def _blockdiag(w, k):
    a, b = w.shape
    out = jnp.zeros((k * a, k * b), dtype=w.dtype)
    for i in range(k):
        out = out.at[i * a:(i + 1) * a, i * b:(i + 1) * b].set(w)
    return out


def kernel(x, edge_index, edge_attr, batch, lin_in_W, lin_in_b, msg_W1, msg_b1,
           msg_g1, msg_be1, msg_W2, msg_b2, msg_g2, msg_be2, upd_W1, upd_b1,
           upd_g1, upd_be1, upd_W2, upd_b2, upd_g2, upd_be2, out_W, out_b):
    L = msg_W1.shape[0]
    src = edge_index[0]
    dst = edge_index[1]

    ms1 = msg_g1 * _BNS
    ms2 = msg_g2 * _BNS
    us1 = upd_g1 * _BNS
    us2 = upd_g2 * _BNS
    mWd = msg_W1[:, :_H, :] * ms1[:, None, :]
    mWs = msg_W1[:, _H:2 * _H, :] * ms1[:, None, :]
    mWe = msg_W1[:, 2 * _H:, :] * ms1[:, None, :]
    mbd = msg_b1 * ms1 + msg_be1
    uWh = upd_W1[:, :_H, :] * us1[:, None, :]
    uWa = upd_W1[:, _H:, :] * us1[:, None, :]
    ub1 = upd_b1 * us1 + upd_be1

    f32 = jnp.float32
    zeros_n = jnp.zeros((_N, _H), dtype=f32)

    wep = [_blockdiag(mWe[l], 8) for l in range(L)]
    w2p = [_blockdiag(msg_W2[l], 8) for l in range(L)]
    b2t = [jnp.tile(msg_b2[l], 8).reshape(1, 8 * _H) for l in range(L)]
    s2t = [jnp.tile(ms2[l], 8).reshape(1, 8 * _H) for l in range(L)]
    be2t = [jnp.tile(msg_be2[l], 8).reshape(1, 8 * _H) for l in range(L)]

    inproj = pl.pallas_call(
        _inproj_body,
        grid=(_N // _RN,),
        in_specs=[_row_spec(_RN, 128), _full_spec((128, _H)), _full_spec((1, _H)),
                  _full_spec((_H, _HL)), _full_spec((1, _HL)), _full_spec((_H, _HL))],
        out_specs=[_row_spec(_RN, _H), _row_spec(_RN, _HL), _row_spec(_RN, _HL)],
        out_shape=[jax.ShapeDtypeStruct((_N, _H), f32),
                   jax.ShapeDtypeStruct((_N, _HL), f32),
                   jax.ShapeDtypeStruct((_N, _HL), f32)],
    )
    h, A, B = inproj(x, lin_in_W, lin_in_b.reshape(1, _H),
                     mWd[0], mbd[0].reshape(1, _HL), mWs[0])

    edge_mlp_p = []
    for p in range(_NP):
        poff = p * (_PE // _EB)
        edge_mlp_p.append(pl.pallas_call(
            _edge_body,
            grid=(_PE // _EB,),
            in_specs=[_row_spec(_BB, 8 * _HL),
                      pl.BlockSpec((_BB, 128), lambda i, poff=poff: (i + poff, 0)),
                      _full_spec((128, 8 * _HL)), _full_spec((8 * _HL, 8 * _H)),
                      _full_spec((1, 8 * _H)), _full_spec((1, 8 * _H)),
                      _full_spec((1, 8 * _H))],
            out_specs=_row_spec(_EB // 2, 2 * _H),
            out_shape=jax.ShapeDtypeStruct((_PE // 2, 2 * _H), f32),
        ))

    def _pspec(j):
        return pl.BlockSpec((_RN, _H), lambda i, j=j: (i + j * (_N // _RN), 0))

    update = pl.pallas_call(
        _update_body,
        grid=(_N // _RN,),
        in_specs=[_row_spec(_RN, _H),
                  _pspec(0), _pspec(1), _pspec(0), _pspec(1),
                  _full_spec((_H, _HL)), _full_spec((_H, _HL)), _full_spec((1, _HL)),
                  _full_spec((_HL, _H)), _full_spec((1, _H)), _full_spec((1, _H)),
                  _full_spec((1, _H)),
                  _full_spec((_H, _HL)), _full_spec((1, _HL)), _full_spec((_H, _HL))],
        out_specs=[_row_spec(_RN, _H), _row_spec(_RN, _HL), _row_spec(_RN, _HL)],
        out_shape=[jax.ShapeDtypeStruct((_N, _H), f32),
                   jax.ShapeDtypeStruct((_N, _HL), f32),
                   jax.ShapeDtypeStruct((_N, _HL), f32)],
    )

    def _fpspec(j):
        return pl.BlockSpec((_N, _H), lambda i, j=j: (j, 0))

    final = pl.pallas_call(
        _final_body,
        grid=(1,),
        in_specs=[_full_spec((_N, _H)),
                  _fpspec(0), _fpspec(1), _fpspec(0), _fpspec(1),
                  _full_spec((_H, _HL)), _full_spec((_H, _HL)), _full_spec((1, _HL)),
                  _full_spec((_HL, _H)), _full_spec((1, _H)), _full_spec((1, _H)),
                  _full_spec((1, _H)),
                  _full_spec((_N, 1)), _full_spec((_H, 1)), _full_spec((1, 1))],
        out_specs=_full_spec((_G, 1)),
        out_shape=jax.ShapeDtypeStruct((_G, 1), f32),
    )

    mesh = plsc.VectorSubcoreMesh(core_axis_name="c", subcore_axis_name="s")
    scp = pltpu.CompilerParams(use_tc_tiling_on_sc=False)

    gather_scratch = [pltpu.VMEM((_WB, _GB), jnp.int32),
                      pltpu.VMEM((_WB, _GB), jnp.int32),
                      pltpu.VMEM((_K, _GB, _HL), f32),
                      pltpu.VMEM((_K, _GB, _HL), f32),
                      pltpu.VMEM((_K, _GR, 8 * _HL), f32),
                      pltpu.VMEM((2, _GB), jnp.int32),
                      pltpu.SemaphoreType.DMA,
                      pltpu.SemaphoreType.DMA]
    scatter_scratch = [pltpu.VMEM((_WB, _GB), jnp.int32),
                       pltpu.VMEM((2, _GB), jnp.int32),
                       pltpu.VMEM((_KS, _GB, _H), f32),
                       pltpu.VMEM_SHARED((_N, _H), f32),
                       pltpu.SemaphoreType.DMA]

    gather_p = []
    scatter_p = []
    for p in range(_NP):
        gather_p.append(pl.kernel(
            _make_gather_body(p * _PB),
            out_type=jax.ShapeDtypeStruct((_PE // 8, 8 * _HL), f32),
            scratch_types=gather_scratch,
            mesh=mesh,
            compiler_params=scp,
        ))
        scatter_p.append(pl.kernel(
            _make_scatter_body(p * _PB),
            out_type=jax.ShapeDtypeStruct((_NC * _N, _H), f32),
            scratch_types=scatter_scratch,
            mesh=mesh,
            compiler_params=scp,
        ))

    src2 = src.reshape(_EBLK, _GB)
    dst2 = dst.reshape(_EBLK, _GB)
    ea8 = edge_attr.reshape(_E // 8, 8 * _DE)

    for l in range(L):
        parts = []
        for p in range(_NP):
            g8 = gather_p[p](A, B, src2, dst2)
            eout8 = edge_mlp_p[p](g8, ea8, wep[l], w2p[l],
                                  b2t[l], s2t[l], be2t[l])
            eout = eout8.reshape(_PE, _H)
            parts.append(scatter_p[p](eout, dst2, zeros_n))
        if l + 1 < L:
            h, A, B = update(h, parts[0], parts[0], parts[1], parts[1],
                             uWh[l], uWa[l], ub1[l].reshape(1, _HL),
                             upd_W2[l], upd_b2[l].reshape(1, _H),
                             us2[l].reshape(1, _H), upd_be2[l].reshape(1, _H),
                             mWd[l + 1], mbd[l + 1].reshape(1, _HL), mWs[l + 1])
        else:
            out = final(h, parts[0], parts[0], parts[1], parts[1],
                        uWh[l], uWa[l], ub1[l].reshape(1, _HL),
                        upd_W2[l], upd_b2[l].reshape(1, _H),
                        us2[l].reshape(1, _H), upd_be2[l].reshape(1, _H),
                        batch.reshape(_N, 1), out_W, out_b.reshape(1, 1))
    return out.reshape(-1)
